# Initial kernel scaffold; baseline (speedup 1.0000x reference)
import jax
import jax.numpy as jnp
from jax.experimental import pallas as pl


def kernel(seq_rep, bppm, xyz_init, rbf_mu, rbf_sigma, thresh, W_node, W_msg, b_msg, w_out):
    raise NotImplementedError("write your pallas kernel here")


# speedup vs baseline: 8.2076x; 8.2076x over previous
import functools

import jax
import jax.numpy as jnp
from jax import lax
from jax.experimental import pallas as pl
from jax.experimental.pallas import tpu as pltpu
from jax.experimental.pallas import tpu_sc as plsc

NC, NS = 2, 16
NW = NC * NS
LN = 4096
RPT = LN // NW
ECAP = 8192
ETOT = NW * ECAP
DAB = 256
DM = 128
NRBF = 30
VL = 16
GCH = 128

_MESH = dict(core_axis_name="c", subcore_axis_name="s", num_cores=NC,
             num_subcores=NS)
_SC_PARAMS = pltpu.CompilerParams(needs_layout_passes=False,
                                  use_tc_tiling_on_sc=True)


def _wid():
    return lax.axis_index("s") * NC + lax.axis_index("c")


def _k1_body(seq_ref, wn_ref, wsd_ref, ab_ref):
    h = jnp.maximum(
        jnp.dot(seq_ref[...], wn_ref[0], preferred_element_type=jnp.float32),
        0.0)
    ab_ref[0] = jnp.dot(h, wsd_ref[0], preferred_element_type=jnp.float32)


def _node_tables(seq_rep, w_node, wsd):
    ts = 512
    return pl.pallas_call(
        _k1_body,
        grid=(2, LN // ts),
        in_specs=[
            pl.BlockSpec((ts, seq_rep.shape[1]), lambda b, t: (t, 0)),
            pl.BlockSpec((1,) + w_node.shape[1:], lambda b, t: (b, 0, 0)),
            pl.BlockSpec((1,) + wsd.shape[1:], lambda b, t: (b, 0, 0)),
        ],
        out_specs=pl.BlockSpec((1, ts, DAB), lambda b, t: (b, t, 0)),
        out_shape=jax.ShapeDtypeStruct((2, LN, DAB), jnp.float32),
    )(seq_rep, w_node, wsd)


def _k2_body(bppm, thr_in, src_o, dst_o, pij_o, et_o, cnt_o,
             rowb, srcb, dstb, pijb, etb, thrb, cntb, pfxb, sem):
    wid = _wid()
    row0 = wid * RPT
    lane = lax.iota(jnp.int32, VL)
    zi = jnp.zeros((VL,), jnp.int32)
    zf = jnp.zeros((VL,), jnp.float32)

    pltpu.sync_copy(thr_in, thrb)
    thr = thrb[...]
    cur00 = jnp.where(wid == NW - 1, RPT - 1, RPT).astype(jnp.int32)

    def zero_body(i, c):
        idx = i * VL + lane
        srcb[pl.ds(i * VL, VL)] = zi
        dstb[pl.ds(i * VL, VL)] = zi
        pijb[pl.ds(i * VL, VL)] = zf
        etb[pl.ds(i * VL, VL)] = jnp.where(idx >= cur00, jnp.ones((VL,), jnp.float32), zf)
        return c

    lax.fori_loop(0, ECAP // VL, zero_body, 0)

    def bb_body(i, c):
        s = row0 + i * VL + lane
        valid = s < (LN - 1)
        srcb[pl.ds(i * VL, VL)] = jnp.where(valid, s, 0)
        dstb[pl.ds(i * VL, VL)] = jnp.where(valid, s + 1, 0)
        pijb[pl.ds(i * VL, VL)] = jnp.ones((VL,), jnp.float32)
        etb[pl.ds(i * VL, VL)] = zf
        return c

    lax.fori_loop(0, RPT // VL, bb_body, 0)
    cur0 = cur00
    NCH = LN // VL
    onei = jnp.ones((VL,), jnp.int32)
    onef = jnp.ones((VL,), jnp.float32)
    dump = ECAP + lane

    def row_body(i, curv):
        absrow = row0 + i
        pltpu.sync_copy(bppm.at[absrow], rowb)

        def ch_body(c, cv):
            v = rowb[pl.ds(c * VL, VL)]
            col = c * VL + lane
            m = (v > thr) & (col > absrow + 2)
            s = plsc.cumsum(onei, mask=m)
            pos = jnp.where(m, cv + s - 1, dump)
            pos = jnp.minimum(pos, dump)
            plsc.store_scatter(srcb, [pos], jnp.full((VL,), absrow, jnp.int32))
            plsc.store_scatter(dstb, [pos], col)
            plsc.store_scatter(pijb, [pos], v)
            return cv + plsc.all_reduce_population_count(m)

        return lax.fori_loop((absrow + 3) // VL, NCH, ch_body, curv)

    curv0 = jnp.full((VL,), 1, jnp.int32) * cur0
    cntv2 = lax.fori_loop(0, RPT, row_body, curv0)
    cntv2 = jnp.minimum(cntv2, ECAP)

    pltpu.sync_copy(srcb.at[pl.ds(0, ECAP)], src_o.at[wid])
    pltpu.sync_copy(dstb.at[pl.ds(0, ECAP)], dst_o.at[wid])
    pltpu.sync_copy(pijb.at[pl.ds(0, ECAP)], pij_o.at[wid])
    pltpu.sync_copy(etb.at[pl.ds(0, ECAP)], et_o.at[wid])
    cntb[...] = cntv2
    pltpu.sync_copy(cntb, cnt_o.at[wid])


def _build_edges(bppm, thresh):
    thr16 = jnp.full((VL,), thresh, jnp.float32)
    f = pl.kernel(
        _k2_body,
        out_type=(
            jax.ShapeDtypeStruct((NW, ECAP), jnp.int32),
            jax.ShapeDtypeStruct((NW, ECAP), jnp.int32),
            jax.ShapeDtypeStruct((NW, ECAP), jnp.float32),
            jax.ShapeDtypeStruct((NW, ECAP), jnp.float32),
            jax.ShapeDtypeStruct((NW, VL), jnp.int32),
        ),
        mesh=plsc.VectorSubcoreMesh(**_MESH),
        compiler_params=_SC_PARAMS,
        scratch_types=[
            pltpu.VMEM((LN,), jnp.float32),
            pltpu.VMEM((ECAP + VL,), jnp.int32),
            pltpu.VMEM((ECAP + VL,), jnp.int32),
            pltpu.VMEM((ECAP + VL,), jnp.float32),
            pltpu.VMEM((ECAP + VL,), jnp.float32),
            pltpu.VMEM((VL,), jnp.float32),
            pltpu.VMEM((VL,), jnp.int32),
            pltpu.VMEM((VL,), jnp.int32),
            pltpu.SemaphoreType.DMA,
        ],
    )
    return f(bppm, thr16)


def _k3_body(ab, xyz, src_i, dst_i, cnt_i, q_o, abs_o, abd_o,
             xyzv, srcv, dstv, qv, cntv, gbufa, gbufb, sema, semb):
    wid = _wid()
    lane = lax.iota(jnp.int32, VL)
    pltpu.sync_copy(xyz, xyzv)
    pltpu.sync_copy(src_i.at[wid], srcv)
    pltpu.sync_copy(dst_i.at[wid], dstv)
    pltpu.sync_copy(cnt_i.at[wid], cntv)
    cnt = jnp.max(cntv[...])

    onei = jnp.full((VL,), 1, jnp.int32)
    twoi = jnp.full((VL,), 2, jnp.int32)

    def q_body(i, c):
        s3 = srcv[pl.ds(i * VL, VL)] * 3
        d3 = dstv[pl.ds(i * VL, VL)] * 3
        rx = plsc.load_gather(xyzv, [d3]) - plsc.load_gather(xyzv, [s3])
        ry = plsc.load_gather(xyzv, [d3 + onei]) - plsc.load_gather(xyzv, [s3 + onei])
        rz = plsc.load_gather(xyzv, [d3 + twoi]) - plsc.load_gather(xyzv, [s3 + twoi])
        qv[pl.ds(i * VL, VL)] = rx * rx + ry * ry + rz * rz
        return c

    lax.fori_loop(0, (cnt + VL - 1) // VL, q_body, 0)
    pltpu.sync_copy(qv, q_o.at[wid])

    def g_body(i, c):
        da = pltpu.async_copy(ab.at[srcv.at[pl.ds(i * GCH, GCH)]], gbufa, sema)
        db = pltpu.async_copy(ab.at[dstv.at[pl.ds(i * GCH, GCH)]], gbufb, semb)
        da.wait()
        pltpu.sync_copy(gbufa, abs_o.at[wid, pl.ds(i * GCH, GCH)])
        db.wait()
        pltpu.sync_copy(gbufb, abd_o.at[wid, pl.ds(i * GCH, GCH)])
        return c

    lax.fori_loop(0, (cnt + GCH - 1) // GCH, g_body, 0)


def _edge_gathers(ab_b, xyz, src, dst, cnt):
    f = pl.kernel(
        _k3_body,
        out_type=(
            jax.ShapeDtypeStruct((NW, ECAP), jnp.float32),
            jax.ShapeDtypeStruct((NW, ECAP, DAB), jnp.float32),
            jax.ShapeDtypeStruct((NW, ECAP, DAB), jnp.float32),
        ),
        mesh=plsc.VectorSubcoreMesh(**_MESH),
        compiler_params=_SC_PARAMS,
        scratch_types=[
            pltpu.VMEM((LN * 3,), jnp.float32),
            pltpu.VMEM((ECAP,), jnp.int32),
            pltpu.VMEM((ECAP,), jnp.int32),
            pltpu.VMEM((ECAP,), jnp.float32),
            pltpu.VMEM((VL,), jnp.int32),
            pltpu.VMEM((GCH, DAB), jnp.float32),
            pltpu.VMEM((GCH, DAB), jnp.float32),
            pltpu.SemaphoreType.DMA,
            pltpu.SemaphoreType.DMA,
        ],
    )
    return f(ab_b, xyz, src, dst, cnt)


def _k4_body(abs_ref, abd_ref, q_ref, pij_ref, et_ref, mu_ref, i2s_ref,
             wes_ref, bm_ref, wo_ref, wf_ref, wr_ref):
    q = jnp.maximum(q_ref[...], 0.0)
    d = jnp.sqrt(q)
    rbf = jnp.exp(-((d - mu_ref[...]) ** 2) * i2s_ref[...])
    es = jnp.concatenate([et_ref[...], pij_ref[...], rbf], axis=1)
    ce = jnp.dot(es, wes_ref[...], preferred_element_type=jnp.float32) \
        + bm_ref[...]
    gs = abs_ref[...]
    gd = abd_ref[...]
    mf = jnp.maximum(gs[:, :DM] + gd[:, DM:] + ce, 0.0)
    mr = jnp.maximum(gd[:, :DM] + gs[:, DM:] + ce, 0.0)
    wo = wo_ref[...]
    wf_ref[...] = jnp.tanh(jnp.dot(mf, wo, preferred_element_type=jnp.float32))
    wr_ref[...] = jnp.tanh(jnp.dot(mr, wo, preferred_element_type=jnp.float32))


def _edge_weights(abs_g, abd_g, q, pij, et, mu, i2s, wes, bm, wo):
    ts = 512
    col = lambda t: (t, 0)
    rep = lambda t: (0, 0)
    return pl.pallas_call(
        _k4_body,
        grid=(ETOT // ts,),
        in_specs=[
            pl.BlockSpec((ts, DAB), col),
            pl.BlockSpec((ts, DAB), col),
            pl.BlockSpec((ts, 1), col),
            pl.BlockSpec((ts, 1), col),
            pl.BlockSpec((ts, 1), col),
            pl.BlockSpec((1, NRBF), rep),
            pl.BlockSpec((1, NRBF), rep),
            pl.BlockSpec((32, DM), rep),
            pl.BlockSpec((1, DM), rep),
            pl.BlockSpec((DM, 1), rep),
        ],
        out_specs=[pl.BlockSpec((ts, 1), col), pl.BlockSpec((ts, 1), col)],
        out_shape=[jax.ShapeDtypeStruct((ETOT, 1), jnp.float32)] * 2,
    )(abs_g, abd_g, q, pij, et, mu, i2s, wes, bm, wo)


ZB = LN * 4 // NS


def _k5_body(xyz, src_i, dst_i, cnt_i, wf_i, wr_i, part_o,
             xyzv, srcv, dstv, wfv, wrv, cntv, rowb, idxb, zbuf, sem, acc):
    cid = lax.axis_index("c")
    sid = lax.axis_index("s")
    wid = sid * NC + cid
    lane = lax.iota(jnp.int32, VL)
    zf = jnp.zeros((VL,), jnp.float32)

    def zz_body(i, c):
        zbuf[pl.ds(i * VL, VL)] = zf
        return c

    lax.fori_loop(0, ZB // VL, zz_body, 0)
    pltpu.sync_copy(zbuf, acc.at[pl.ds(sid * ZB, ZB)])
    plsc.subcore_barrier()

    pltpu.sync_copy(xyz, xyzv)
    pltpu.sync_copy(src_i.at[wid], srcv)
    pltpu.sync_copy(dst_i.at[wid], dstv)
    pltpu.sync_copy(wf_i.at[wid], wfv)
    pltpu.sync_copy(wr_i.at[wid], wrv)
    pltpu.sync_copy(cnt_i.at[wid], cntv)
    cnt = jnp.max(cntv[...])

    onei = jnp.full((VL,), 1, jnp.int32)
    twoi = jnp.full((VL,), 2, jnp.int32)
    l4 = lane * 4
    r64 = jnp.full((VL,), 4 * VL, jnp.int32)

    def e_body(i, c):
        s = srcv[pl.ds(i * VL, VL)]
        d = dstv[pl.ds(i * VL, VL)]
        wf = wfv[pl.ds(i * VL, VL)]
        wr = wrv[pl.ds(i * VL, VL)]
        valid = (i * VL + lane) < cnt
        s3 = s * 3
        d3 = d * 3
        rx = plsc.load_gather(xyzv, [d3]) - plsc.load_gather(xyzv, [s3])
        ry = plsc.load_gather(xyzv, [d3 + onei]) - plsc.load_gather(xyzv, [s3 + onei])
        rz = plsc.load_gather(xyzv, [d3 + twoi]) - plsc.load_gather(xyzv, [s3 + twoi])
        one = jnp.ones((VL,), jnp.float32)
        zero = jnp.zeros((VL,), jnp.float32)
        d4 = jnp.where(valid, d, 0) * 4
        s4 = jnp.where(valid, s, 0) * 4
        vals_f = (rx * wf, ry * wf, rz * wf, one)
        vals_r = (-rx * wr, -ry * wr, -rz * wr, one)
        for k in range(4):
            kk = jnp.full((VL,), k, jnp.int32)
            plsc.store_scatter(rowb, [l4 + kk], jnp.where(valid, vals_f[k], zero))
            plsc.store_scatter(idxb, [l4 + kk], d4 + kk)
            plsc.store_scatter(rowb, [r64 + l4 + kk], jnp.where(valid, vals_r[k], zero))
            plsc.store_scatter(idxb, [r64 + l4 + kk], s4 + kk)
        pltpu.sync_copy(rowb, acc.at[idxb], add=True)
        return c

    lax.fori_loop(0, (cnt + VL - 1) // VL, e_body, 0)
    plsc.subcore_barrier()

    @pl.when(sid == 0)
    def _():
        pltpu.sync_copy(acc, part_o.at[cid])


def _scatter_agg(xyz, src, dst, cnt, wf, wr):
    f = pl.kernel(
        _k5_body,
        out_type=jax.ShapeDtypeStruct((NC, LN * 4), jnp.float32),
        mesh=plsc.VectorSubcoreMesh(**_MESH),
        compiler_params=_SC_PARAMS,
        scratch_types=[
            pltpu.VMEM((LN * 3,), jnp.float32),
            pltpu.VMEM((ECAP,), jnp.int32),
            pltpu.VMEM((ECAP,), jnp.int32),
            pltpu.VMEM((ECAP,), jnp.float32),
            pltpu.VMEM((ECAP,), jnp.float32),
            pltpu.VMEM((VL,), jnp.int32),
            pltpu.VMEM((8 * VL,), jnp.float32),
            pltpu.VMEM((8 * VL,), jnp.int32),
            pltpu.VMEM((ZB,), jnp.float32),
            pltpu.SemaphoreType.DMA,
            pltpu.VMEM_SHARED((LN * 4,), jnp.float32),
        ],
    )
    return f(xyz, src, dst, cnt, wf, wr)


def _k6_body(xyz_ref, part_ref, out_ref):
    p = part_ref[0] + part_ref[1]
    out_ref[...] = xyz_ref[...] + p[:, :3] / (p[:, 3:4] + 1.0)


def _apply_update(xyz, part):
    return pl.pallas_call(
        _k6_body,
        out_shape=jax.ShapeDtypeStruct((LN, 3), jnp.float32),
    )(xyz, part)


def kernel(seq_rep, bppm, xyz_init, rbf_mu, rbf_sigma, thresh,
           W_node, W_msg, b_msg, w_out):
    nb = W_node.shape[0]
    dh = W_node.shape[2]
    wsd = jnp.concatenate([W_msg[:, :dh, :], W_msg[:, dh:2 * dh, :]], axis=2)
    wes = W_msg[:, 2 * dh:, :]
    mu = rbf_mu.reshape(1, NRBF)
    i2s = (1.0 / (2.0 * rbf_sigma * rbf_sigma)).reshape(1, NRBF)

    ab = _node_tables(seq_rep, W_node, wsd)
    src, dst, pij, et, cnt = _build_edges(bppm, thresh)
    pij2 = pij.reshape(ETOT, 1)
    et2 = et.reshape(ETOT, 1)

    xyz = xyz_init
    for b in range(nb):
        xyzf = xyz.reshape(LN * 3)
        q, abs_g, abd_g = _edge_gathers(ab[b], xyzf, src, dst, cnt)
        wf, wr = _edge_weights(
            abs_g.reshape(ETOT, DAB), abd_g.reshape(ETOT, DAB),
            q.reshape(ETOT, 1), pij2, et2, mu, i2s,
            wes[b], b_msg[b].reshape(1, DM), w_out[b])
        part = _scatter_agg(xyzf, src, dst, cnt,
                            wf.reshape(NW, ECAP), wr.reshape(NW, ECAP))
        xyz = _apply_update(xyz, part.reshape(NC, LN, 4))
    return xyz
